# baseline (speedup 1.0000x reference)
import jax
import jax.numpy as jnp
from jax.experimental import pallas as pl
from jax.experimental.pallas import tpu as pltpu

_NUM_CHUNKS = 8


def _copy_kernel(in_hbm, out_hbm, buf, in_sems, out_sems):
    rows, _ = in_hbm.shape
    chunk = rows // _NUM_CHUNKS

    def copy_in(i):
        return pltpu.make_async_copy(
            in_hbm.at[pl.ds(i * chunk, chunk)], buf.at[i], in_sems.at[i])

    def copy_out(i):
        return pltpu.make_async_copy(
            buf.at[i], out_hbm.at[pl.ds(i * chunk, chunk)], out_sems.at[i])

    for i in range(_NUM_CHUNKS):
        copy_in(i).start(priority=0)
    for i in range(_NUM_CHUNKS):
        copy_in(i).wait()
        copy_out(i).start(priority=1)
    for i in range(_NUM_CHUNKS):
        copy_out(i).wait()


def kernel(inputs, embedding_table):
    del embedding_table
    rows, cols = inputs.shape
    chunk = rows // _NUM_CHUNKS
    return pl.pallas_call(
        _copy_kernel,
        out_shape=jax.ShapeDtypeStruct(inputs.shape, inputs.dtype),
        in_specs=[pl.BlockSpec(memory_space=pl.ANY)],
        out_specs=pl.BlockSpec(memory_space=pl.ANY),
        scratch_shapes=[
            pltpu.VMEM((_NUM_CHUNKS, chunk, cols), inputs.dtype),
            pltpu.SemaphoreType.DMA((_NUM_CHUNKS,)),
            pltpu.SemaphoreType.DMA((_NUM_CHUNKS,)),
        ],
    )(inputs)


# speedup vs baseline: 1.0341x; 1.0341x over previous
import jax
import jax.numpy as jnp
from jax.experimental import pallas as pl
from jax.experimental.pallas import tpu as pltpu

_NUM_CHUNKS = 8


def _copy_kernel(in_hbm, out_hbm, buf, in_sems, out_sems):
    rows, _ = in_hbm.shape
    chunk = rows // _NUM_CHUNKS

    def copy_in(i):
        return pltpu.make_async_copy(
            in_hbm.at[pl.ds(i * chunk, chunk)], buf.at[i], in_sems.at[i])

    def copy_out(i):
        return pltpu.make_async_copy(
            buf.at[i], out_hbm.at[pl.ds(i * chunk, chunk)], out_sems.at[i])

    for i in range(_NUM_CHUNKS):
        copy_in(i).start(priority=i % 2)
    for i in range(_NUM_CHUNKS):
        copy_in(i).wait()
        copy_out(i).start(priority=i % 2)
    for i in range(_NUM_CHUNKS):
        copy_out(i).wait()


def kernel(inputs, embedding_table):
    del embedding_table
    rows, cols = inputs.shape
    chunk = rows // _NUM_CHUNKS
    return pl.pallas_call(
        _copy_kernel,
        out_shape=jax.ShapeDtypeStruct(inputs.shape, inputs.dtype),
        in_specs=[pl.BlockSpec(memory_space=pl.ANY)],
        out_specs=pl.BlockSpec(memory_space=pl.ANY),
        scratch_shapes=[
            pltpu.VMEM((_NUM_CHUNKS, chunk, cols), inputs.dtype),
            pltpu.SemaphoreType.DMA((_NUM_CHUNKS,)),
            pltpu.SemaphoreType.DMA((_NUM_CHUNKS,)),
        ],
    )(inputs)
